# baseline (speedup 1.0000x reference)
import functools

import jax
import jax.numpy as jnp
from jax.experimental import pallas as pl
from jax.experimental.pallas import tpu as pltpu

_BLK = 4096


def _vq_block_kernel(x_ref, cb_ref, ek_ref, ids_ref, cbs_ref, c2_ref, cba_ref):
    @pl.when(pl.program_id(0) == 0)
    def _init():
        cbi = cb_ref[...]
        kk = cbi.shape[0]
        cbs_ref[...] = -2.0 * cbi
        c2_ref[...] = jnp.sum(cbi * cbi, axis=1, keepdims=True)
        cba_ref[:, :64] = cbi
        cba_ref[:, 64:65] = jax.lax.broadcasted_iota(
            jnp.int32, (kk, 1), 0).astype(jnp.float32)
        cba_ref[:, 65:66] = jnp.ones((kk, 1), jnp.float32)

    x = x_ref[...]
    k = cbs_ref.shape[0]
    scoresT = jax.lax.dot_general(
        cbs_ref[...], x, (((1,), (1,)), ((), ())),
        preferred_element_type=jnp.float32,
    )
    distT = scoresT + c2_ref[...]
    mind = jnp.min(distT, axis=0)
    onehotT = (distT <= mind[None, :]).astype(jnp.float32)
    ek_aug = jax.lax.dot_general(
        onehotT, cba_ref[...], (((0,), (0,)), ((), ())),
        preferred_element_type=jnp.float32,
    )
    ek_ref[...] = ek_aug[:, :64]
    ids_ref[...] = ek_aug[:, 64:65].astype(jnp.int32)
    count = ek_aug[:, 65:66]


@functools.partial(jax.jit, static_argnames=())
def kernel(x, codebook):
    n, d = x.shape
    k = codebook.shape[0]
    grid = (n // _BLK,)
    ek, ids = pl.pallas_call(
        _vq_block_kernel,
        grid=grid,
        in_specs=[
            pl.BlockSpec((_BLK, d), lambda i: (i, 0)),
            pl.BlockSpec((k, d), lambda i: (0, 0)),
        ],
        out_specs=[
            pl.BlockSpec((_BLK, d), lambda i: (i, 0)),
            pl.BlockSpec((_BLK, 1), lambda i: (i, 0)),
        ],
        out_shape=[
            jax.ShapeDtypeStruct((n, d), jnp.float32),
            jax.ShapeDtypeStruct((n, 1), jnp.int32),
        ],
        scratch_shapes=[
            pltpu.VMEM((k, d), jnp.float32),
            pltpu.VMEM((k, 1), jnp.float32),
            pltpu.VMEM((k, 66), jnp.float32),
        ],
    )(x, codebook)
    return (ek, ids.reshape(n))


# speedup vs baseline: 1.0209x; 1.0209x over previous
import functools

import jax
import jax.numpy as jnp
from jax.experimental import pallas as pl
from jax.experimental.pallas import tpu as pltpu

_BLK = 4096


def _vq_block_kernel(xa_ref, cb_ref, ek_ref, ids_ref, cbs_ref, cba_ref):
    @pl.when(pl.program_id(0) == 0)
    def _init():
        cbi = cb_ref[...]
        kk = cbi.shape[0]
        cbs_ref[:, :64] = -2.0 * cbi
        cbs_ref[:, 64:65] = jnp.sum(cbi * cbi, axis=1, keepdims=True)
        cba_ref[:, :64] = cbi
        cba_ref[:, 64:65] = jax.lax.broadcasted_iota(
            jnp.int32, (kk, 1), 0).astype(jnp.float32)
        cba_ref[:, 65:66] = jnp.ones((kk, 1), jnp.float32)

    xa = xa_ref[...]
    k = cbs_ref.shape[0]
    distT = jax.lax.dot_general(
        cbs_ref[...], xa, (((1,), (1,)), ((), ())),
        preferred_element_type=jnp.float32,
    )
    mind = jnp.min(distT, axis=0)
    onehotT = (distT <= mind[None, :]).astype(jnp.float32)
    ek_aug = jax.lax.dot_general(
        onehotT, cba_ref[...], (((0,), (0,)), ((), ())),
        preferred_element_type=jnp.float32,
    )
    ek_ref[...] = ek_aug[:, :64]
    ids_ref[...] = ek_aug[:, 64:65].astype(jnp.int32)
    count = ek_aug[:, 65:66]

    @pl.when(jnp.max(count) > 1.5)
    def _fix_ties():
        code_iota = jax.lax.broadcasted_iota(jnp.int32, distT.shape, 0)
        ids_t = jnp.min(
            jnp.where(distT <= mind[None, :], code_iota, k), axis=0
        ).astype(jnp.int32)
        oh = (code_iota == ids_t[None, :]).astype(jnp.float32)
        ek_ref[...] = jax.lax.dot_general(
            oh, cba_ref[:, :64], (((0,), (0,)), ((), ())),
            preferred_element_type=jnp.float32,
        )
        ids_ref[...] = ids_t[:, None]


@functools.partial(jax.jit, static_argnames=())
def kernel(x, codebook):
    n, d = x.shape
    k = codebook.shape[0]
    xa = jnp.concatenate([x, jnp.ones((n, 1), jnp.float32)], axis=1)
    grid = (n // _BLK,)
    ek, ids = pl.pallas_call(
        _vq_block_kernel,
        grid=grid,
        in_specs=[
            pl.BlockSpec((_BLK, d + 1), lambda i: (i, 0)),
            pl.BlockSpec((k, d), lambda i: (0, 0)),
        ],
        out_specs=[
            pl.BlockSpec((_BLK, d), lambda i: (i, 0)),
            pl.BlockSpec((_BLK, 1), lambda i: (i, 0)),
        ],
        out_shape=[
            jax.ShapeDtypeStruct((n, d), jnp.float32),
            jax.ShapeDtypeStruct((n, 1), jnp.int32),
        ],
        scratch_shapes=[
            pltpu.VMEM((k, 65), jnp.float32),
            pltpu.VMEM((k, 66), jnp.float32),
        ],
    )(xa, codebook)
    return (ek, ids.reshape(n))
